# baseline (speedup 1.0000x reference)
import functools

import jax
import jax.numpy as jnp
from jax import lax
from jax.experimental import pallas as pl
from jax.experimental.pallas import tpu as pltpu
from jax.experimental.pallas import tpu_sc as plsc

K = 1024
D = 64
BETA = 0.25
N = 8192
BLK = 4096
NBLK = N // BLK

NC = 2
NS = 16
NW = NC * NS
RPW = N // NW
CH = 128
NCH = RPW // CH


def _main_body(z_ref, cbt_ref, idx_ref, loss_ref, perp_ref,
               c2_s, hist_s, lsum_s):
    i = pl.program_id(0)
    cb = lax.transpose(cbt_ref[...], (1, 0))

    @pl.when(i == 0)
    def _init():
        c2 = jnp.sum(cb * cb, axis=1)
        c2_s[...] = c2[None, :]
        hist_s[...] = jnp.zeros((1, K), jnp.float32)
        lsum_s[0] = 0.0

    zt = z_ref[...]
    z = jnp.concatenate(
        [lax.transpose(zt[b], (1, 0)) for b in range(8)], axis=0)
    s = lax.dot_general(z, cb, (((1,), (1,)), ((), ())),
                        preferred_element_type=jnp.float32)
    z2 = jnp.sum(z * z, axis=1, keepdims=True)
    d = z2 + c2_s[...] - 2.0 * s
    mind = jnp.min(d, axis=1)
    iota1 = lax.broadcasted_iota(jnp.int32, (1, K), 1)
    idx = jnp.min(jnp.where(d == mind[:, None], iota1, K), axis=1)
    idx_ref[...] = idx.reshape(8, BLK // 8)
    lsum_s[0] += jnp.sum(mind)
    eqf = (idx[:, None] == iota1).astype(jnp.float32)
    ones_row = jnp.ones((1, BLK), jnp.float32)
    counts_blk = lax.dot_general(ones_row, eqf, (((1,), (0,)), ((), ())),
                                 preferred_element_type=jnp.float32)
    hist_s[...] += counts_blk

    @pl.when(i == NBLK - 1)
    def _fin():
        m = lsum_s[0] / float(N * D)
        counts = hist_s[...]
        e = counts * (1.0 / float(N))
        ent = jnp.sum(e * jnp.log(e + 1e-10))
        loss_ref[...] = jnp.full((1, 1), m + BETA * m, jnp.float32)
        perp_ref[...] = jnp.full((1, 1), jnp.exp(-ent), jnp.float32)


def _tc_main(z_t, codebook):
    tok = BLK // 8
    return pl.pallas_call(
        _main_body,
        grid=(NBLK,),
        in_specs=[
            pl.BlockSpec((8, D, tok), lambda i: (0, 0, i)),
            pl.BlockSpec((D, K), lambda i: (0, 0)),
        ],
        out_specs=[
            pl.BlockSpec((8, tok), lambda i: (0, i)),
            pl.BlockSpec((1, 1), lambda i: (0, 0)),
            pl.BlockSpec((1, 1), lambda i: (0, 0)),
        ],
        out_shape=[
            jax.ShapeDtypeStruct((8, 1024), jnp.int32),
            jax.ShapeDtypeStruct((1, 1), jnp.float32),
            jax.ShapeDtypeStruct((1, 1), jnp.float32),
        ],
        scratch_shapes=[
            pltpu.VMEM((1, K), jnp.float32),
            pltpu.VMEM((1, K), jnp.float32),
            pltpu.SMEM((1,), jnp.float32),
        ],
    )(z_t, codebook)


def _sc_gather_body(cb_hbm, idx_hbm, zq_hbm, idx_v, rows_v, sem):
    wid = lax.axis_index("s") * NC + lax.axis_index("c")
    base = wid * RPW
    b = wid // (1024 // RPW)
    t0 = (wid % (1024 // RPW)) * RPW
    del base
    for c in range(NCH):
        pltpu.sync_copy(idx_hbm.at[b, pl.ds(t0 + c * CH, CH)], idx_v.at[c])
    copies = [pltpu.async_copy(cb_hbm.at[idx_v.at[c]], rows_v.at[c], sem)
              for c in range(NCH)]
    for c in range(NCH):
        copies[c].wait()
        pltpu.sync_copy(rows_v.at[c], zq_hbm.at[b, pl.ds(t0 + c * CH, CH)])


def _sc_gather(codebook, idx2):
    mesh = plsc.VectorSubcoreMesh(core_axis_name="c", subcore_axis_name="s")
    f = functools.partial(
        pl.kernel,
        mesh=mesh,
        out_type=jax.ShapeDtypeStruct((8, 1024, D), jnp.float32),
        scratch_types=[
            pltpu.VMEM((NCH, CH), jnp.int32),
            pltpu.VMEM((NCH, CH, D), jnp.float32),
            pltpu.SemaphoreType.DMA,
        ],
        compiler_params=pltpu.CompilerParams(
            needs_layout_passes=False, use_tc_tiling_on_sc=False),
    )(_sc_gather_body)
    return f(codebook, idx2)


def kernel(z, codebook):
    z_t = jnp.transpose(z, (0, 2, 1))
    indices, loss2, perp2 = _tc_main(z_t, codebook.T)
    z_q_st = _sc_gather(codebook, indices)
    return (loss2[0, 0], z_q_st, perp2[0, 0], indices)


# speedup vs baseline: 1.0013x; 1.0013x over previous
import functools

import jax
import jax.numpy as jnp
from jax import lax
from jax.experimental import pallas as pl
from jax.experimental.pallas import tpu as pltpu
from jax.experimental.pallas import tpu_sc as plsc

K = 1024
D = 64
BETA = 0.25
N = 8192
BLK = 4096
NBLK = N // BLK

NC = 2
NS = 16
NW = NC * NS
RPW = N // NW
CH = 128
NCH = RPW // CH


def _main_body(z_ref, cb_ref, idx_ref, loss_ref, perp_ref,
               c2_s, hist_s, lsum_s):
    i = pl.program_id(0)
    cb = cb_ref[...]

    @pl.when(i == 0)
    def _init():
        c2 = jnp.sum(cb * cb, axis=1)
        c2_s[...] = c2[None, :]
        hist_s[...] = jnp.zeros((1, K), jnp.float32)
        lsum_s[0] = 0.0

    zt = z_ref[...]
    z = jnp.concatenate(
        [lax.transpose(zt[b], (1, 0)) for b in range(8)], axis=0)
    s = lax.dot_general(z, cb, (((1,), (1,)), ((), ())),
                        preferred_element_type=jnp.float32)
    z2 = jnp.sum(z * z, axis=1, keepdims=True)
    d = z2 + c2_s[...] - 2.0 * s
    mind = jnp.min(d, axis=1)
    iota1 = lax.broadcasted_iota(jnp.int32, (1, K), 1)
    idx = jnp.min(jnp.where(d == mind[:, None], iota1, K), axis=1)
    idx_ref[...] = idx.reshape(8, BLK // 8)
    lsum_s[0] += jnp.sum(mind)
    eqf = (idx[:, None] == iota1).astype(jnp.float32)
    ones_row = jnp.ones((1, BLK), jnp.float32)
    counts_blk = lax.dot_general(ones_row, eqf, (((1,), (0,)), ((), ())),
                                 preferred_element_type=jnp.float32)
    hist_s[...] += counts_blk

    @pl.when(i == NBLK - 1)
    def _fin():
        m = lsum_s[0] / float(N * D)
        counts = hist_s[...]
        e = counts * (1.0 / float(N))
        ent = jnp.sum(e * jnp.log(e + 1e-10))
        loss_ref[...] = jnp.full((1, 1), m + BETA * m, jnp.float32)
        perp_ref[...] = jnp.full((1, 1), jnp.exp(-ent), jnp.float32)


def _tc_main(z_t, codebook):
    tok = BLK // 8
    return pl.pallas_call(
        _main_body,
        grid=(NBLK,),
        in_specs=[
            pl.BlockSpec((8, D, tok), lambda i: (0, 0, i)),
            pl.BlockSpec((K, D), lambda i: (0, 0)),
        ],
        out_specs=[
            pl.BlockSpec((8, tok), lambda i: (0, i)),
            pl.BlockSpec((1, 1), lambda i: (0, 0)),
            pl.BlockSpec((1, 1), lambda i: (0, 0)),
        ],
        out_shape=[
            jax.ShapeDtypeStruct((8, 1024), jnp.int32),
            jax.ShapeDtypeStruct((1, 1), jnp.float32),
            jax.ShapeDtypeStruct((1, 1), jnp.float32),
        ],
        scratch_shapes=[
            pltpu.VMEM((1, K), jnp.float32),
            pltpu.VMEM((1, K), jnp.float32),
            pltpu.SMEM((1,), jnp.float32),
        ],
    )(z_t, codebook)


def _sc_gather_body(cb_hbm, idx_hbm, zq_hbm, idx_v, rows_v, sem):
    wid = lax.axis_index("s") * NC + lax.axis_index("c")
    base = wid * RPW
    b = wid // (1024 // RPW)
    t0 = (wid % (1024 // RPW)) * RPW
    del base
    for c in range(NCH):
        pltpu.sync_copy(idx_hbm.at[b, pl.ds(t0 + c * CH, CH)], idx_v.at[c])
    copies = [pltpu.async_copy(cb_hbm.at[idx_v.at[c]], rows_v.at[c], sem)
              for c in range(NCH)]
    for c in range(NCH):
        copies[c].wait()
        pltpu.sync_copy(rows_v.at[c], zq_hbm.at[b, pl.ds(t0 + c * CH, CH)])


def _sc_gather(codebook, idx2):
    mesh = plsc.VectorSubcoreMesh(core_axis_name="c", subcore_axis_name="s")
    f = functools.partial(
        pl.kernel,
        mesh=mesh,
        out_type=jax.ShapeDtypeStruct((8, 1024, D), jnp.float32),
        scratch_types=[
            pltpu.VMEM((NCH, CH), jnp.int32),
            pltpu.VMEM((NCH, CH, D), jnp.float32),
            pltpu.SemaphoreType.DMA,
        ],
        compiler_params=pltpu.CompilerParams(
            needs_layout_passes=False, use_tc_tiling_on_sc=False),
    )(_sc_gather_body)
    return f(codebook, idx2)


def kernel(z, codebook):
    z_t = jnp.transpose(z, (0, 2, 1))
    indices, loss2, perp2 = _tc_main(z_t, codebook)
    z_q_st = _sc_gather(codebook, indices)
    return (loss2[0, 0], z_q_st, perp2[0, 0], indices)
